# Initial kernel scaffold; baseline (speedup 1.0000x reference)
import functools

import jax
import jax.numpy as jnp
from jax import lax
from jax.experimental import pallas as pl
from jax.experimental.pallas import tpu as pltpu
from jax.experimental.pallas import tpu_sc as plsc

NUM_ROWS = 1_000_000
D = 32
B = 16_384
NC, NS = 2, 16
NW = NC * NS
IDX_COLS = 128
IDX_ROWS = B // IDX_COLS
ROWS_PW = IDX_ROWS // NW

_sc_mesh = plsc.VectorSubcoreMesh(
    core_axis_name="c", subcore_axis_name="s", num_cores=NC, num_subcores=NS)


def _worker_id():
  return lax.axis_index("s") * NC + lax.axis_index("c")


@functools.partial(
    pl.kernel,
    out_type=jax.ShapeDtypeStruct((IDX_ROWS, IDX_COLS, D), jnp.float32),
    mesh=_sc_mesh,
    scratch_types=[
        pltpu.VMEM((ROWS_PW, IDX_COLS), jnp.int32),
        pltpu.VMEM((ROWS_PW, IDX_COLS, D), jnp.float32),
        pltpu.SemaphoreType.DMA,
    ],
)
def _sc_gather(mem_hbm, idx_hbm, out_hbm, idx_v, rows_v, sem):
  w = _worker_id()
  r0 = w * ROWS_PW
  pltpu.sync_copy(idx_hbm.at[pl.ds(r0, ROWS_PW)], idx_v)
  handles = [
      pltpu.async_copy(mem_hbm.at[idx_v.at[j]], rows_v.at[j], sem)
      for j in range(ROWS_PW)
  ]
  for h in handles:
    h.wait()
  pltpu.sync_copy(rows_v, out_hbm.at[pl.ds(r0, ROWS_PW)])


_BLK = 2048
_NBLK = B // _BLK


def _blend_loss_body(l_ref, told_ref, tg_ref, loss_ref):
  i = pl.program_id(0)
  l = l_ref[...]
  tg = 0.9 * told_ref[...] + 0.1 * jax.nn.sigmoid(l)
  tg_ref[...] = tg
  part = jnp.sum(jnp.maximum(l, 0.0) - l * tg + jnp.log1p(jnp.exp(-jnp.abs(l))))

  @pl.when(i == 0)
  def _():
    loss_ref[0, 0] = 0.0

  loss_ref[0, 0] += part

  @pl.when(i == _NBLK - 1)
  def _():
    loss_ref[0, 0] = loss_ref[0, 0] / (B * D)


_blend_loss = pl.pallas_call(
    _blend_loss_body,
    grid=(_NBLK,),
    in_specs=[
        pl.BlockSpec((_BLK, D), lambda i: (i, 0)),
        pl.BlockSpec((_BLK, D), lambda i: (i, 0)),
    ],
    out_specs=[
        pl.BlockSpec((_BLK, D), lambda i: (i, 0)),
        pl.BlockSpec(memory_space=pltpu.SMEM),
    ],
    out_shape=[
        jax.ShapeDtypeStruct((B, D), jnp.float32),
        jax.ShapeDtypeStruct((1, 1), jnp.float32),
    ],
)


_CP_ROWS = 8000
_CP_GRID = NUM_ROWS // _CP_ROWS


def _copy_body(x_ref, o_ref):
  o_ref[...] = x_ref[...]


_bank_copy = pl.pallas_call(
    _copy_body,
    grid=(_CP_GRID,),
    in_specs=[pl.BlockSpec((_CP_ROWS, D), lambda i: (i, 0))],
    out_specs=pl.BlockSpec((_CP_ROWS, D), lambda i: (i, 0)),
    out_shape=jax.ShapeDtypeStruct((NUM_ROWS, D), jnp.float32),
)


@functools.partial(
    pl.kernel,
    out_type=(),
    mesh=_sc_mesh,
    scratch_types=[
        pltpu.VMEM((ROWS_PW, IDX_COLS), jnp.int32),
        pltpu.VMEM((ROWS_PW, IDX_COLS, D), jnp.float32),
        pltpu.SemaphoreType.DMA,
    ],
)
def _sc_scatter(mem_hbm, idx_hbm, tgt_hbm, idx_v, rows_v, sem):
  w = _worker_id()
  r0 = w * ROWS_PW
  pltpu.sync_copy(idx_hbm.at[pl.ds(r0, ROWS_PW)], idx_v)
  pltpu.sync_copy(tgt_hbm.at[pl.ds(r0, ROWS_PW)], rows_v)
  handles = [
      pltpu.async_copy(rows_v.at[j], mem_hbm.at[idx_v.at[j]], sem)
      for j in range(ROWS_PW)
  ]
  for h in handles:
    h.wait()


def kernel(logits, labels, index, targets_mem):
  del labels
  idx2d = index.reshape(IDX_ROWS, IDX_COLS)
  t_old = _sc_gather(targets_mem, idx2d).reshape(B, D)
  targets, loss = _blend_loss(logits, t_old)
  mem2 = _bank_copy(targets_mem)
  mem_ref = jax.new_ref(mem2)
  _sc_scatter(mem_ref, idx2d, targets.reshape(IDX_ROWS, IDX_COLS, D))
  new_mem = mem_ref[...]
  return loss[0, 0], new_mem


# speedup vs baseline: 1.1576x; 1.1576x over previous
import functools

import jax
import jax.numpy as jnp
from jax import lax
from jax.experimental import pallas as pl
from jax.experimental.pallas import tpu as pltpu
from jax.experimental.pallas import tpu_sc as plsc

NUM_ROWS = 1_000_000
D = 32
B = 16_384
NC, NS = 2, 16
NW = NC * NS
IDX_COLS = 128
IDX_ROWS = B // IDX_COLS
ROWS_PW = IDX_ROWS // NW

def _worker_id():
  return lax.axis_index("s") * NC + lax.axis_index("c")


@functools.cache
def _sc_kernels():
  mesh = plsc.VectorSubcoreMesh(
      core_axis_name="c", subcore_axis_name="s", num_cores=NC, num_subcores=NS)
  params = pltpu.CompilerParams(use_tc_tiling_on_sc=False)
  scratch = [
      pltpu.VMEM((ROWS_PW, IDX_COLS), jnp.int32),
      pltpu.VMEM((ROWS_PW, IDX_COLS, D), jnp.float32),
      pltpu.SemaphoreType.DMA,
  ]

  @functools.partial(
      pl.kernel,
      out_type=jax.ShapeDtypeStruct((IDX_ROWS, IDX_COLS, D), jnp.float32),
      mesh=mesh,
      scratch_types=scratch,
      compiler_params=params,
  )
  def sc_gather(mem_hbm, idx_hbm, out_hbm, idx_v, rows_v, sem):
    w = _worker_id()
    r0 = w * ROWS_PW
    pltpu.sync_copy(idx_hbm.at[pl.ds(r0, ROWS_PW)], idx_v)
    handles = [
        pltpu.async_copy(mem_hbm.at[idx_v.at[j]], rows_v.at[j], sem)
        for j in range(ROWS_PW)
    ]
    for h in handles:
      h.wait()
    pltpu.sync_copy(rows_v, out_hbm.at[pl.ds(r0, ROWS_PW)])

  @functools.partial(pl.kernel, out_type=(), mesh=mesh, scratch_types=scratch,
                     compiler_params=params)
  def sc_scatter(mem_hbm, idx_hbm, tgt_hbm, idx_v, rows_v, sem):
    w = _worker_id()
    r0 = w * ROWS_PW
    pltpu.sync_copy(idx_hbm.at[pl.ds(r0, ROWS_PW)], idx_v)
    pltpu.sync_copy(tgt_hbm.at[pl.ds(r0, ROWS_PW)], rows_v)
    handles = [
        pltpu.async_copy(rows_v.at[j], mem_hbm.at[idx_v.at[j]], sem)
        for j in range(ROWS_PW)
    ]
    for h in handles:
      h.wait()

  return sc_gather, sc_scatter


_BLK = 2048
_NBLK = B // _BLK


def _blend_loss_body(l_ref, told_ref, tg_ref, loss_ref):
  i = pl.program_id(0)
  l = l_ref[...]
  tg = 0.9 * told_ref[...] + 0.1 * jax.nn.sigmoid(l)
  tg_ref[...] = tg
  part = jnp.sum(jnp.maximum(l, 0.0) - l * tg + jnp.log1p(jnp.exp(-jnp.abs(l))))

  @pl.when(i == 0)
  def _():
    loss_ref[0, 0] = 0.0

  loss_ref[0, 0] += part

  @pl.when(i == _NBLK - 1)
  def _():
    loss_ref[0, 0] = loss_ref[0, 0] / (B * D)


_blend_loss = pl.pallas_call(
    _blend_loss_body,
    grid=(_NBLK,),
    in_specs=[
        pl.BlockSpec((_BLK, D), lambda i: (i, 0)),
        pl.BlockSpec((_BLK, D), lambda i: (i, 0)),
    ],
    out_specs=[
        pl.BlockSpec((_BLK, D), lambda i: (i, 0)),
        pl.BlockSpec(memory_space=pltpu.SMEM),
    ],
    out_shape=[
        jax.ShapeDtypeStruct((B, D), jnp.float32),
        jax.ShapeDtypeStruct((1, 1), jnp.float32),
    ],
)


_CP_ROWS = 8000
_CP_GRID = NUM_ROWS // _CP_ROWS


def _copy_body(x_ref, o_ref):
  o_ref[...] = x_ref[...]


_bank_copy = pl.pallas_call(
    _copy_body,
    grid=(_CP_GRID,),
    in_specs=[pl.BlockSpec((_CP_ROWS, D), lambda i: (i, 0))],
    out_specs=pl.BlockSpec((_CP_ROWS, D), lambda i: (i, 0)),
    out_shape=jax.ShapeDtypeStruct((NUM_ROWS, D), jnp.float32),
)


def kernel(logits, labels, index, targets_mem):
  del labels
  _sc_gather, _sc_scatter = _sc_kernels()
  idx2d = index.reshape(IDX_ROWS, IDX_COLS)
  t_old = _sc_gather(targets_mem, idx2d).reshape(B, D)
  targets, loss = _blend_loss(logits, t_old)
  mem2 = _bank_copy(targets_mem)
  mem_ref = jax.new_ref(mem2)
  _sc_scatter(mem_ref, idx2d, targets.reshape(IDX_ROWS, IDX_COLS, D))
  new_mem = mem_ref[...]
  return loss[0, 0], new_mem
